# baseline (speedup 1.0000x reference)
import functools

import jax
import jax.numpy as jnp
from jax import lax
from jax.experimental import pallas as pl
from jax.experimental.pallas import tpu as pltpu
from jax.experimental.pallas import tpu_sc as plsc

SEQ = 200
BATCH = 4096
EMB = 64
NC = 2
NS = 16
NW = NC * NS

K = 128
NCHUNK = SEQ


def _gather_body(x_hbm, tpairs_hbm, out_hbm, idx_v, gbuf, obuf, gsem):
    wid = lax.axis_index("s") * NC + lax.axis_index("c")
    b0 = wid * K
    pltpu.sync_copy(x_hbm.at[:, pl.ds(b0, K)], idx_v)
    iota = lax.iota(jnp.int32, 16)

    def chunk(j, _):
        pltpu.async_copy(tpairs_hbm.at[idx_v.at[j]], gbuf, gsem).wait()
        for g in range(8):
            rows = iota + g * 16

            def kbody(k, _):
                vals = plsc.load_gather(gbuf, [rows, jnp.full((16,), 0, jnp.int32) + k])
                obuf[k, pl.ds(g * 16, 16)] = vals
                return 0

            lax.fori_loop(0, EMB, kbody, 0)
        pltpu.sync_copy(obuf, out_hbm.at[j, :, pl.ds(b0, K)])
        return 0

    lax.fori_loop(0, NCHUNK, chunk, 0)


@jax.jit
def kernel(x, table):
    x32 = x.astype(jnp.int32)
    tpairs = jnp.pad(table, ((0, 0), (0, 64)))
    out = pl.kernel(
        _gather_body,
        out_type=jax.ShapeDtypeStruct((SEQ, EMB, BATCH), jnp.float32),
        mesh=plsc.VectorSubcoreMesh(core_axis_name="c", subcore_axis_name="s"),
        scratch_types=[
            pltpu.VMEM((NCHUNK, K), jnp.int32),
            pltpu.VMEM((K, 128), jnp.float32),
            pltpu.VMEM((EMB, K), jnp.float32),
            pltpu.SemaphoreType.DMA,
        ],
        compiler_params=pltpu.CompilerParams(
            use_tc_tiling_on_sc=True, needs_layout_passes=False
        ),
    )(x32, tpairs)
    return out.transpose(0, 2, 1)


# speedup vs baseline: 2.4738x; 2.4738x over previous
import functools

import jax
import jax.numpy as jnp
from jax import lax
from jax.experimental import pallas as pl
from jax.experimental.pallas import tpu as pltpu
from jax.experimental.pallas import tpu_sc as plsc

SEQ = 200
BATCH = 4096
EMB = 64
NC = 2
NS = 16
NW = NC * NS

K = 128
NCHUNK = SEQ
NBUF = 4
LOOKAHEAD = 2


def _gather_body(x_hbm, tpad_hbm, out_hbm, idx_v, bufs, gsem, osem):
    wid = lax.axis_index("s") * NC + lax.axis_index("c")
    b0 = wid * K
    pltpu.sync_copy(x_hbm.at[:, pl.ds(b0, K)], idx_v)

    def start_gather(j, b):
        pltpu.async_copy(tpad_hbm.at[idx_v.at[j]], bufs.at[b], gsem.at[b])

    def wait_gather(j, b):
        pltpu.make_async_copy(tpad_hbm.at[idx_v.at[j]], bufs.at[b],
                              gsem.at[b]).wait()

    def start_write(j, b):
        pltpu.async_copy(bufs.at[b], out_hbm.at[pl.ds(j * BATCH + b0, K)],
                         osem.at[b])

    def wait_write(j, b):
        pltpu.make_async_copy(bufs.at[b], out_hbm.at[pl.ds(j * BATCH + b0, K)],
                              osem.at[b]).wait()

    for b in range(LOOKAHEAD):
        start_gather(b, b)

    def outer(g, _):
        for bi in range(NBUF):
            j = g * NBUF + bi
            bn = (bi + LOOKAHEAD) % NBUF
            jp = j + LOOKAHEAD - NBUF

            @pl.when(jp >= 0)
            def _():
                wait_write(jp, bn)

            @pl.when(j + LOOKAHEAD < NCHUNK)
            def _():
                start_gather(j + LOOKAHEAD, bn)

            wait_gather(j, bi)
            start_write(j, bi)
        return 0

    lax.fori_loop(0, NCHUNK // NBUF, outer, 0)

    for t in range(LOOKAHEAD):
        j = NCHUNK - LOOKAHEAD + t
        wait_write(j, j % NBUF)


@jax.jit
def kernel(x, table):
    x32 = x.astype(jnp.int32)
    tpad = jnp.pad(table, ((0, 0), (0, 64)))
    out = pl.kernel(
        _gather_body,
        out_type=jax.ShapeDtypeStruct((SEQ * BATCH, 128), jnp.float32),
        mesh=plsc.VectorSubcoreMesh(core_axis_name="c", subcore_axis_name="s"),
        scratch_types=[
            pltpu.VMEM((NCHUNK, K), jnp.int32),
            pltpu.VMEM((NBUF, K, 128), jnp.float32),
            pltpu.SemaphoreType.DMA((NBUF,)),
            pltpu.SemaphoreType.DMA((NBUF,)),
        ],
        compiler_params=pltpu.CompilerParams(
            use_tc_tiling_on_sc=True, needs_layout_passes=False
        ),
    )(x32, tpad)
    return out[:, :EMB].reshape(SEQ, BATCH, EMB)
